# Initial kernel scaffold; baseline (speedup 1.0000x reference)
import functools

import jax
import jax.numpy as jnp
from jax import lax
from jax.experimental import pallas as pl
from jax.experimental.pallas import tpu as pltpu
from jax.experimental.pallas import tpu_sc as plsc

N = 10000
IN_DIM = 128
HID = 64

NC = 2
NS = 16
NW = NC * NS
CHUNK = 128
NP = 10016
RPS = NP // NS
DEG_LANES = 16

_mesh = plsc.VectorSubcoreMesh(core_axis_name="c", subcore_axis_name="s")


def _deg_body(dst_hbm, ones_hbm, zeros_hbm, out_hbm, dst_v, ones_v, acc_sh):
    c = lax.axis_index("c")
    s = lax.axis_index("s")
    wid = c * NS + s
    nch = dst_hbm.shape[1]
    pltpu.sync_copy(dst_hbm.at[wid], dst_v)
    pltpu.sync_copy(ones_hbm, ones_v)
    r0 = s * RPS
    pltpu.sync_copy(zeros_hbm.at[pl.ds(r0, RPS)], acc_sh.at[pl.ds(r0, RPS)])
    plsc.subcore_barrier()

    def body(j, carry):
        pltpu.sync_copy(ones_v, acc_sh.at[dst_v.at[j]], add=True)
        return carry

    lax.fori_loop(0, nch, body, 0)
    plsc.subcore_barrier()
    pltpu.sync_copy(acc_sh.at[pl.ds(r0, RPS)], out_hbm.at[c].at[pl.ds(r0, RPS)])


def _agg_body(src_hbm, dst_hbm, g_hbm, zeros_hbm, out_hbm,
              src_v, dst_v, buf_v, acc_sh, sem):
    c = lax.axis_index("c")
    s = lax.axis_index("s")
    wid = c * NS + s
    nch = src_hbm.shape[1]
    pltpu.sync_copy(src_hbm.at[wid], src_v)
    pltpu.sync_copy(dst_hbm.at[wid], dst_v)
    r0 = s * RPS
    pltpu.sync_copy(zeros_hbm.at[pl.ds(r0, RPS)], acc_sh.at[pl.ds(r0, RPS)])
    plsc.subcore_barrier()

    def body(j, carry):
        pltpu.async_copy(g_hbm.at[src_v.at[j]], buf_v, sem).wait()
        pltpu.sync_copy(buf_v, acc_sh.at[dst_v.at[j]], add=True)
        return carry

    lax.fori_loop(0, nch, body, 0)
    plsc.subcore_barrier()
    pltpu.sync_copy(acc_sh.at[pl.ds(r0, RPS)], out_hbm.at[c].at[pl.ds(r0, RPS)])


def _mm_body(x_ref, w_ref, degp_ref, g_ref):
    degp = degp_ref[...]
    deg = degp[0, :, 0] + degp[1, :, 0] + 1.0
    dinv = lax.rsqrt(deg)
    h = jnp.dot(x_ref[...], w_ref[...], preferred_element_type=jnp.float32)
    g_ref[...] = h * dinv[:, None]


def _fin_body(accp_ref, g_ref, degp_ref, b_ref, gam_ref, bet_ref, out_ref):
    degp = degp_ref[...]
    deg = degp[0, :, 0] + degp[1, :, 0] + 1.0
    dinv = lax.rsqrt(deg)
    accp = accp_ref[...]
    v = (accp[0] + accp[1] + g_ref[...]) * dinv[:, None] + b_ref[0]
    mu = jnp.mean(v, axis=1, keepdims=True)
    var = jnp.mean((v - mu) ** 2, axis=1, keepdims=True)
    y = (v - mu) * lax.rsqrt(var + 1e-5) * gam_ref[0] + bet_ref[0]
    out_ref[...] = jnp.maximum(y, 0.0)


def kernel(x, ei, W, b, ln_gamma, ln_beta):
    E = ei.shape[1]
    src = ei[0].astype(jnp.int32)
    dst = ei[1].astype(jnp.int32)

    nch = -(-E // (NW * CHUNK))
    EP = NW * nch * CHUNK
    src_p = jnp.concatenate([src, jnp.zeros((EP - E,), jnp.int32)])
    dst_p = jnp.concatenate([dst, jnp.full((EP - E,), N, jnp.int32)])
    src3 = src_p.reshape(NW, nch, CHUNK)
    dst3 = dst_p.reshape(NW, nch, CHUNK)

    ones_chunk = jnp.ones((CHUNK, DEG_LANES), jnp.float32)
    zeros_deg = jnp.zeros((NP, DEG_LANES), jnp.float32)
    zeros_acc = jnp.zeros((NP, HID), jnp.float32)

    deg_call = pl.kernel(
        _deg_body, mesh=_mesh,
        out_type=jax.ShapeDtypeStruct((NC, NP, DEG_LANES), jnp.float32),
        scratch_types=[
            pltpu.VMEM((nch, CHUNK), jnp.int32),
            pltpu.VMEM((CHUNK, DEG_LANES), jnp.float32),
            pltpu.VMEM_SHARED((NP, DEG_LANES), jnp.float32),
        ],
    )
    degp = deg_call(dst3, ones_chunk, zeros_deg)

    BR = 1000
    g = pl.pallas_call(
        _mm_body,
        grid=(N // BR,),
        in_specs=[
            pl.BlockSpec((BR, IN_DIM), lambda i: (i, 0)),
            pl.BlockSpec((IN_DIM, HID), lambda i: (0, 0)),
            pl.BlockSpec((NC, BR, DEG_LANES), lambda i: (0, i, 0)),
        ],
        out_specs=pl.BlockSpec((BR, HID), lambda i: (i, 0)),
        out_shape=jax.ShapeDtypeStruct((N, HID), jnp.float32),
    )(x, W, degp)

    agg_call = pl.kernel(
        _agg_body, mesh=_mesh,
        out_type=jax.ShapeDtypeStruct((NC, NP, HID), jnp.float32),
        scratch_types=[
            pltpu.VMEM((nch, CHUNK), jnp.int32),
            pltpu.VMEM((nch, CHUNK), jnp.int32),
            pltpu.VMEM((CHUNK, HID), jnp.float32),
            pltpu.VMEM_SHARED((NP, HID), jnp.float32),
            pltpu.SemaphoreType.DMA,
        ],
    )
    accp = agg_call(src3, dst3, g, zeros_acc)

    out = pl.pallas_call(
        _fin_body,
        grid=(N // BR,),
        in_specs=[
            pl.BlockSpec((NC, BR, HID), lambda i: (0, i, 0)),
            pl.BlockSpec((BR, HID), lambda i: (i, 0)),
            pl.BlockSpec((NC, BR, DEG_LANES), lambda i: (0, i, 0)),
            pl.BlockSpec((1, HID), lambda i: (0, 0)),
            pl.BlockSpec((1, HID), lambda i: (0, 0)),
            pl.BlockSpec((1, HID), lambda i: (0, 0)),
        ],
        out_specs=pl.BlockSpec((BR, HID), lambda i: (i, 0)),
        out_shape=jax.ShapeDtypeStruct((N, HID), jnp.float32),
    )(accp, g, degp, b.reshape(1, HID), ln_gamma.reshape(1, HID),
      ln_beta.reshape(1, HID))
    return out


# speedup vs baseline: 26.1858x; 26.1858x over previous
import functools

import jax
import jax.numpy as jnp
from jax import lax
from jax.experimental import pallas as pl
from jax.experimental.pallas import tpu as pltpu
from jax.experimental.pallas import tpu_sc as plsc

N = 10000
IN_DIM = 128
HID = 64

NC = 2
NS = 16
NW = NC * NS
CHUNK = 128
NP = 10112
RPS = NP // NS
DEG_LANES = 16

_mesh = plsc.VectorSubcoreMesh(core_axis_name="c", subcore_axis_name="s")


def _deg_body(dst_hbm, ones_hbm, zeros_hbm, out_hbm, dst_v, ones_v, acc_sh):
    c = lax.axis_index("c")
    s = lax.axis_index("s")
    wid = c * NS + s
    nch = dst_hbm.shape[1]
    pltpu.sync_copy(dst_hbm.at[wid], dst_v)
    pltpu.sync_copy(ones_hbm, ones_v)
    r0 = s * RPS
    pltpu.sync_copy(zeros_hbm.at[pl.ds(r0, RPS)], acc_sh.at[pl.ds(r0, RPS)])
    plsc.subcore_barrier()

    def body(j, carry):
        pltpu.sync_copy(ones_v, acc_sh.at[dst_v.at[j]], add=True)
        return carry

    lax.fori_loop(0, nch, body, 0)
    plsc.subcore_barrier()
    pltpu.sync_copy(acc_sh.at[pl.ds(r0, RPS)], out_hbm.at[c].at[pl.ds(r0, RPS)])


def _agg_body(src_hbm, dst_hbm, g_hbm, zeros_hbm, out_hbm,
              src_v, dst_v, buf_v, acc_sh, sem):
    c = lax.axis_index("c")
    s = lax.axis_index("s")
    wid = c * NS + s
    nch = src_hbm.shape[1]
    pltpu.sync_copy(src_hbm.at[wid], src_v)
    pltpu.sync_copy(dst_hbm.at[wid], dst_v)
    r0 = s * RPS
    pltpu.sync_copy(zeros_hbm.at[pl.ds(r0, RPS)], acc_sh.at[pl.ds(r0, RPS)])
    plsc.subcore_barrier()

    def body(j, carry):
        pltpu.async_copy(g_hbm.at[src_v.at[j]], buf_v, sem).wait()
        pltpu.sync_copy(buf_v, acc_sh.at[dst_v.at[j]], add=True)
        return carry

    lax.fori_loop(0, nch, body, 0)
    plsc.subcore_barrier()
    pltpu.sync_copy(acc_sh.at[pl.ds(r0, RPS)], out_hbm.at[c].at[pl.ds(r0, RPS)])


def _mm_body(x_ref, w_ref, degp_ref, g_ref):
    degp = degp_ref[...]
    deg = degp[0, :, 0] + degp[1, :, 0] + 1.0
    dinv = lax.rsqrt(deg)
    h = jnp.dot(x_ref[...], w_ref[...], preferred_element_type=jnp.float32)
    g_ref[...] = h * dinv[:, None]


def _fin_body(accp_ref, g_ref, degp_ref, b_ref, gam_ref, bet_ref, out_ref):
    degp = degp_ref[...]
    deg = degp[0, :, 0] + degp[1, :, 0] + 1.0
    dinv = lax.rsqrt(deg)
    accp = accp_ref[...]
    v = (accp[0] + accp[1] + g_ref[...]) * dinv[:, None] + b_ref[0]
    mu = jnp.mean(v, axis=1, keepdims=True)
    var = jnp.mean((v - mu) ** 2, axis=1, keepdims=True)
    y = (v - mu) * lax.rsqrt(var + 1e-5) * gam_ref[0] + bet_ref[0]
    out_ref[...] = jnp.maximum(y, 0.0)


def kernel(x, ei, W, b, ln_gamma, ln_beta):
    E = ei.shape[1]
    src = ei[0].astype(jnp.int32)
    dst = ei[1].astype(jnp.int32)

    nch = -(-E // (NW * CHUNK))
    EP = NW * nch * CHUNK
    src_p = jnp.concatenate([src, jnp.zeros((EP - E,), jnp.int32)])
    dst_p = jnp.concatenate([dst, jnp.full((EP - E,), N, jnp.int32)])
    src3 = src_p.reshape(NW, nch, CHUNK)
    dst3 = dst_p.reshape(NW, nch, CHUNK)

    ones_chunk = jnp.ones((CHUNK, DEG_LANES), jnp.float32)
    zeros_deg = jnp.zeros((NP, DEG_LANES), jnp.float32)
    zeros_acc = jnp.zeros((NP, HID), jnp.float32)

    deg_call = pl.kernel(
        _deg_body, mesh=_mesh,
        compiler_params=pltpu.CompilerParams(use_tc_tiling_on_sc=False),
        out_type=jax.ShapeDtypeStruct((NC, NP, DEG_LANES), jnp.float32),
        scratch_types=[
            pltpu.VMEM((nch, CHUNK), jnp.int32),
            pltpu.VMEM((CHUNK, DEG_LANES), jnp.float32),
            pltpu.VMEM_SHARED((NP, DEG_LANES), jnp.float32),
        ],
    )
    degp = deg_call(dst3, ones_chunk, zeros_deg)

    BR = 1000
    g = pl.pallas_call(
        _mm_body,
        grid=(N // BR,),
        in_specs=[
            pl.BlockSpec((BR, IN_DIM), lambda i: (i, 0)),
            pl.BlockSpec((IN_DIM, HID), lambda i: (0, 0)),
            pl.BlockSpec((NC, BR, DEG_LANES), lambda i: (0, i, 0)),
        ],
        out_specs=pl.BlockSpec((BR, HID), lambda i: (i, 0)),
        out_shape=jax.ShapeDtypeStruct((N, HID), jnp.float32),
    )(x, W, degp)

    agg_call = pl.kernel(
        _agg_body, mesh=_mesh,
        compiler_params=pltpu.CompilerParams(use_tc_tiling_on_sc=False),
        out_type=jax.ShapeDtypeStruct((NC, NP, HID), jnp.float32),
        scratch_types=[
            pltpu.VMEM((nch, CHUNK), jnp.int32),
            pltpu.VMEM((nch, CHUNK), jnp.int32),
            pltpu.VMEM((CHUNK, HID), jnp.float32),
            pltpu.VMEM_SHARED((NP, HID), jnp.float32),
            pltpu.SemaphoreType.DMA,
        ],
    )
    accp = agg_call(src3, dst3, g, zeros_acc)

    out = pl.pallas_call(
        _fin_body,
        grid=(N // BR,),
        in_specs=[
            pl.BlockSpec((NC, BR, HID), lambda i: (0, i, 0)),
            pl.BlockSpec((BR, HID), lambda i: (i, 0)),
            pl.BlockSpec((NC, BR, DEG_LANES), lambda i: (0, i, 0)),
            pl.BlockSpec((1, HID), lambda i: (0, 0)),
            pl.BlockSpec((1, HID), lambda i: (0, 0)),
            pl.BlockSpec((1, HID), lambda i: (0, 0)),
        ],
        out_specs=pl.BlockSpec((BR, HID), lambda i: (i, 0)),
        out_shape=jax.ShapeDtypeStruct((N, HID), jnp.float32),
    )(accp, g, degp, b.reshape(1, HID), ln_gamma.reshape(1, HID),
      ln_beta.reshape(1, HID))
    return out
